# Initial kernel scaffold; baseline (speedup 1.0000x reference)
import jax
import jax.numpy as jnp
from jax.experimental import pallas as pl


def kernel(inputs, targets):
    raise NotImplementedError("write your pallas kernel here")


# speedup vs baseline: 4.7381x; 4.7381x over previous
import functools

import jax
import jax.numpy as jnp
from jax.scipy.special import gammaln
from jax.experimental import pallas as pl
from jax.experimental.pallas import tpu as pltpu

_C = 20
_VAR = 1.0
_EPS = 1e-16


def _soft_table():
    n = jnp.float32(_C - 1)
    ks = jnp.arange(_C, dtype=jnp.float32)
    ps = ks / n
    eps = jnp.float32(1e-5)
    zero = jnp.float32(0.0)
    mu = ks
    alpha = jnp.sqrt(jnp.maximum(mu * (1.0 - ps) - _VAR, zero)
                     / (jnp.maximum(mu, eps) * (1.0 + mu / jnp.maximum(n - mu, eps))))
    mu_p = mu[:, None, None]
    ks_p = ks[None, :, None]
    i_p = ks[None, None, :]
    ps2 = jnp.stack([ps + alpha, ps - mu * alpha / jnp.maximum(n - mu, eps)], axis=0)
    valid = jnp.logical_and(i_p <= mu_p, i_p >= mu_p + ks_p - n)
    validf = valid.astype(jnp.float32)
    binomials = jnp.exp(
        gammaln(n - mu_p + 1.0) + gammaln(mu_p + 1.0)
        - gammaln(jnp.maximum(ks_p - i_p + 1.0, 1.0))
        - gammaln(i_p + 1.0)
        - gammaln(jnp.maximum(mu_p - i_p + 1.0, 1.0))
        - gammaln(jnp.maximum(n - mu_p - ks_p + i_p + 1.0, 1.0))
    ) * validf
    p = ps2[:, :, None, None]
    stable = jnp.logical_not(jnp.logical_or(jnp.isclose(p, 0.0), jnp.isclose(p, 1.0)))
    sn = stable.astype(jnp.float32)
    p = jnp.where(stable, p, 0.5)
    products = jnp.exp(
        (jnp.log(p[0]) * i_p
         + jnp.log(1.0 - p[0]) * (mu_p - i_p)
         + jnp.log(p[1]) * (ks_p - i_p) * sn[0]
         + jnp.log(1.0 - p[1]) * (n - mu_p - ks_p + i_p))
        * sn[1] * validf
    )
    return (binomials * products).sum(axis=-1)


def _body(x_ref, t_ref, sw_ref, o_ref, acc_ref, *, inv_b):
    i = pl.program_id(0)

    @pl.when(i == 0)
    def _init():
        acc_ref[0, 0] = jnp.float32(0.0)

    x = x_ref[...]
    t = t_ref[...]
    logx = jnp.log(x + jnp.float32(_EPS))
    onehot = (jax.lax.broadcasted_iota(jnp.int32, (t.shape[0], 32), 1)
              == t).astype(jnp.float32)
    gath = jax.lax.dot_general(
        onehot, sw_ref[...], (((1,), (0,)), ((), ())),
        preferred_element_type=jnp.float32)
    acc_ref[0, 0] += jnp.sum(logx * gath[:, :_C])

    @pl.when(i == pl.num_programs(0) - 1)
    def _fin():
        o_ref[0, 0] = -acc_ref[0, 0] * jnp.float32(inv_b) - jnp.float32(1.0)


def kernel(inputs, targets):
    b = inputs.shape[0]
    blk = 8192
    grid = b // blk
    sw = _soft_table()
    swpad = jnp.zeros((32, 128), jnp.float32).at[:_C, :_C].set(sw)
    t2 = targets.astype(jnp.int32).reshape(b, 1)
    out = pl.pallas_call(
        functools.partial(_body, inv_b=1.0 / b),
        grid=(grid,),
        in_specs=[
            pl.BlockSpec((blk, _C), lambda i: (i, 0)),
            pl.BlockSpec((blk, 1), lambda i: (i, 0)),
            pl.BlockSpec((32, 128), lambda i: (0, 0)),
        ],
        out_specs=pl.BlockSpec(memory_space=pltpu.SMEM),
        out_shape=jax.ShapeDtypeStruct((1, 1), jnp.float32),
        scratch_shapes=[pltpu.SMEM((1, 1), jnp.float32)],
        compiler_params=pltpu.CompilerParams(
            dimension_semantics=("arbitrary",)),
    )(inputs, t2, swpad)
    return out[0, 0]
